# Initial kernel scaffold; baseline (speedup 1.0000x reference)
import jax
import jax.numpy as jnp
from jax.experimental import pallas as pl


def kernel(x, expert_bias, gate_w):
    raise NotImplementedError("write your pallas kernel here")


# speedup vs baseline: 1.2577x; 1.2577x over previous
import functools

import jax
import jax.numpy as jnp
from jax.experimental import pallas as pl
from jax.experimental.pallas import tpu as pltpu

DIM = 2048
NUM_EXPERTS = 64
NUM_GROUPS = 8
GROUP_SIZE = 8
NUM_LIMITED_GROUPS = 4
TOP_K = 8
ROUTE_SCALE = 2.5
NEG_INF = float("-inf")


def _router_body(x_ref, w_ref, b_ref, tv_ref, ti_ref, h_ref):
    bt = x_ref.shape[0]
    logits = jax.lax.dot_general(
        x_ref[...], w_ref[...], (((1,), (1,)), ((), ())),
        preferred_element_type=jnp.float32)
    scores = jax.nn.sigmoid(logits)
    sfc = scores + b_ref[...]

    sg = sfc.reshape(bt, NUM_GROUPS, GROUP_SIZE)
    v1 = jnp.max(sg, axis=2)
    iota3 = jax.lax.broadcasted_iota(jnp.int32, sg.shape, 2)
    is1 = sg == v1[:, :, None]
    a1 = jnp.min(jnp.where(is1, iota3, GROUP_SIZE), axis=2)
    sg2 = jnp.where(iota3 == a1[:, :, None], NEG_INF, sg)
    v2 = jnp.max(sg2, axis=2)
    gs = v1 + v2

    ga = gs[:, None, :]
    gb = gs[:, :, None]
    ih = jax.lax.broadcasted_iota(jnp.int32, (bt, NUM_GROUPS, NUM_GROUPS), 2)
    ig = jax.lax.broadcasted_iota(jnp.int32, (bt, NUM_GROUPS, NUM_GROUPS), 1)
    beats = (ga > gb) | ((ga == gb) & (ih < ig))
    rank = jnp.sum(beats.astype(jnp.int32), axis=2)
    keep = (rank < NUM_LIMITED_GROUPS).astype(jnp.float32)
    keep64 = jnp.broadcast_to(keep[:, :, None],
                              (bt, NUM_GROUPS, GROUP_SIZE)).reshape(bt, NUM_EXPERTS)
    m = jnp.where(keep64 > 0.0, sfc, NEG_INF)

    iota_l = jax.lax.broadcasted_iota(jnp.int32, (bt, NUM_EXPERTS), 1)
    vals, idxs = [], []
    hacc = jnp.zeros((1, NUM_EXPERTS), jnp.float32)
    for _ in range(TOP_K):
        mx = jnp.max(m, axis=1, keepdims=True)
        idx = jnp.min(jnp.where(m == mx, iota_l, NUM_EXPERTS), axis=1,
                      keepdims=True)
        onehot = iota_l == idx
        val = jnp.sum(jnp.where(onehot, scores, 0.0), axis=1, keepdims=True)
        vals.append(val)
        idxs.append(idx)
        hacc = hacc + jnp.sum(onehot.astype(jnp.float32), axis=0, keepdims=True)
        m = jnp.where(onehot, NEG_INF, m)

    tv = jnp.concatenate(vals, axis=1)
    ti = jnp.concatenate(idxs, axis=1)
    denom = jnp.sum(tv, axis=1, keepdims=True) + 1e-20
    tv_ref[...] = tv / denom * ROUTE_SCALE
    ti_ref[...] = ti

    @pl.when(pl.program_id(0) == 0)
    def _():
        h_ref[...] = jnp.zeros_like(h_ref)

    h_ref[...] += hacc


def kernel(x, expert_bias, gate_w):
    t = x.shape[0]
    bt = 512
    grid = t // bt
    bias2d = expert_bias.reshape(1, NUM_EXPERTS)
    tv, ti, h = pl.pallas_call(
        _router_body,
        grid=(grid,),
        in_specs=[
            pl.BlockSpec((bt, DIM), lambda i: (i, 0)),
            pl.BlockSpec((NUM_EXPERTS, DIM), lambda i: (0, 0)),
            pl.BlockSpec((1, NUM_EXPERTS), lambda i: (0, 0)),
        ],
        out_specs=[
            pl.BlockSpec((bt, TOP_K), lambda i: (i, 0)),
            pl.BlockSpec((bt, TOP_K), lambda i: (i, 0)),
            pl.BlockSpec((1, NUM_EXPERTS), lambda i: (0, 0)),
        ],
        out_shape=[
            jax.ShapeDtypeStruct((t, TOP_K), jnp.float32),
            jax.ShapeDtypeStruct((t, TOP_K), jnp.int32),
            jax.ShapeDtypeStruct((1, NUM_EXPERTS), jnp.float32),
        ],
    )(x, gate_w, bias2d)
    return tv, ti, h.reshape(NUM_EXPERTS)


# speedup vs baseline: 4.2645x; 3.3906x over previous
import functools

import jax
import jax.numpy as jnp
from jax import lax
from jax.experimental import pallas as pl
from jax.experimental.pallas import tpu as pltpu
from jax.experimental.pallas import tpu_sc as plsc

DIM = 2048
NUM_EXPERTS = 64
NUM_GROUPS = 8
GROUP_SIZE = 8
NUM_LIMITED_GROUPS = 4
TOP_K = 8
ROUTE_SCALE = 2.5
NEG_INF = float("-inf")

T = 16384
NS = 16
NW = 32
TPW = T // NW
L = 16
CH = 128
NCH = TPW // CH
CSTEPS = CH // L


def _gate_body(x_ref, w_ref, b_ref, s_ref, sb_ref):
    logits = jax.lax.dot_general(
        w_ref[...], x_ref[...], (((1,), (1,)), ((), ())),
        preferred_element_type=jnp.float32)
    scores = jax.nn.sigmoid(logits)
    s_ref[...] = scores.reshape(s_ref.shape)
    sb_ref[...] = (scores + b_ref[...]).reshape(sb_ref.shape)


def _gate_scores(x, gate_w, expert_bias):
    return pl.pallas_call(
        _gate_body,
        grid=(NW,),
        in_specs=[
            pl.BlockSpec((TPW, DIM), lambda i: (i, 0)),
            pl.BlockSpec((NUM_EXPERTS, DIM), lambda i: (0, 0)),
            pl.BlockSpec((NUM_EXPERTS, 1), lambda i: (0, 0)),
        ],
        out_specs=[
            pl.BlockSpec((1, NUM_EXPERTS, TPW), lambda i: (i, 0, 0)),
            pl.BlockSpec((1, NUM_EXPERTS, TPW), lambda i: (i, 0, 0)),
        ],
        out_shape=[
            jax.ShapeDtypeStruct((NW, NUM_EXPERTS, TPW), jnp.float32),
            jax.ShapeDtypeStruct((NW, NUM_EXPERTS, TPW), jnp.float32),
        ],
    )(x, gate_w, expert_bias.reshape(NUM_EXPERTS, 1))


def _tourney(pairs):
    while len(pairs) > 1:
        nxt = []
        for i in range(0, len(pairs) - 1, 2):
            (vl, il), (vr, ir) = pairs[i], pairs[i + 1]
            cond = vl >= vr
            nxt.append((jnp.where(cond, vl, vr), jnp.where(cond, il, ir)))
        if len(pairs) % 2:
            nxt.append(pairs[-1])
        pairs = nxt
    return pairs[0]


def _sc_router_body(scores_hbm, biased_hbm, outv_hbm, outi_hbm, hist_hbm,
                    sblk, bblk, wrk, outv, outi, cnts, cloc):
    wid = lax.axis_index("s") * 2 + lax.axis_index("c")

    lanes = lax.iota(jnp.int32, L)
    zeros = jnp.zeros((L,), jnp.float32)
    ones = jnp.ones((L,), jnp.float32)
    ninf = jnp.full((L,), NEG_INF, jnp.float32)
    sixteen = jnp.full((L,), L, jnp.int32)
    lanes64 = lanes * jnp.full((L,), NUM_EXPERTS, jnp.int32)
    for c in range(L * NUM_EXPERTS // L):
        cnts[pl.ds(c * L, L)] = zeros

    def step(i, cols):
        sl = pl.ds(i * L, L)

        gsum = []
        for g in range(NUM_GROUPS):
            a = b = None
            for j in range(GROUP_SIZE):
                v = bblk[g * GROUP_SIZE + j, sl]
                if a is None:
                    a, b = v, ninf
                else:
                    b = jnp.maximum(b, jnp.minimum(a, v))
                    a = jnp.maximum(a, v)
            gsum.append(a + b)

        keep = []
        for g in range(NUM_GROUPS):
            rank = None
            for h in range(NUM_GROUPS):
                if h == g:
                    continue
                beats = (gsum[h] >= gsum[g]) if h < g else (gsum[h] > gsum[g])
                r = beats.astype(jnp.int32)
                rank = r if rank is None else rank + r
            keep.append(rank < NUM_LIMITED_GROUPS)

        for e in range(NUM_EXPERTS):
            wrk[e, :] = jnp.where(keep[e // GROUP_SIZE], bblk[e, sl], ninf)

        svals, sidxs = [], []
        for _r in range(TOP_K):
            pairs = [(wrk[e, :], jnp.full((L,), e, jnp.int32))
                     for e in range(NUM_EXPERTS)]
            _v, idx = _tourney(pairs)
            sval = plsc.load_gather(sblk, [idx, cols])
            plsc.store_scatter(wrk, [idx, lanes], ninf)
            plsc.addupdate_scatter(cnts, [lanes64 + idx], ones)
            svals.append(sval)
            sidxs.append(idx)

        ssum = svals[0]
        for r in range(1, TOP_K):
            ssum = ssum + svals[r]
        scale = ROUTE_SCALE / (ssum + 1e-20)
        for r in range(TOP_K):
            rcol = jnp.full((L,), r, jnp.int32)
            plsc.store_scatter(outv, [cols, rcol], svals[r] * scale)
            plsc.store_scatter(outi, [cols, rcol], sidxs[r])
        return cols + sixteen

    base = wid * TPW

    def chunk(c, carry):
        pltpu.sync_copy(scores_hbm.at[wid, :, pl.ds(c * CH, CH)], sblk)
        pltpu.sync_copy(biased_hbm.at[wid, :, pl.ds(c * CH, CH)], bblk)
        lax.fori_loop(0, CSTEPS, step, lanes)
        pltpu.sync_copy(outv, outv_hbm.at[pl.ds(base + c * CH, CH)])
        pltpu.sync_copy(outi, outi_hbm.at[pl.ds(base + c * CH, CH)])
        return carry

    lax.fori_loop(0, NCH, chunk, 0)

    for c in range(NUM_EXPERTS // L):
        acc = cnts[pl.ds(c * L, L)]
        for l in range(1, L):
            acc = acc + cnts[pl.ds(l * NUM_EXPERTS + c * L, L)]
        cloc[pl.ds(c * L, L)] = acc

    pltpu.sync_copy(cloc, hist_hbm.at[wid])


@functools.partial(
    pl.kernel,
    mesh=plsc.VectorSubcoreMesh(core_axis_name="c", subcore_axis_name="s"),
    compiler_params=pltpu.CompilerParams(needs_layout_passes=False),
    out_type=[
        jax.ShapeDtypeStruct((T, TOP_K), jnp.float32),
        jax.ShapeDtypeStruct((T, TOP_K), jnp.int32),
        jax.ShapeDtypeStruct((NW, NUM_EXPERTS), jnp.float32),
    ],
    scratch_types=[
        pltpu.VMEM((NUM_EXPERTS, CH), jnp.float32),
        pltpu.VMEM((NUM_EXPERTS, CH), jnp.float32),
        pltpu.VMEM((NUM_EXPERTS, L), jnp.float32),
        pltpu.VMEM((CH, TOP_K), jnp.float32),
        pltpu.VMEM((CH, TOP_K), jnp.int32),
        pltpu.VMEM((L * NUM_EXPERTS,), jnp.float32),
        pltpu.VMEM((NUM_EXPERTS,), jnp.float32),
    ],
)
def _sc_router(scores_hbm, biased_hbm, outv_hbm, outi_hbm, hist_hbm,
               sblk, bblk, wrk, outv, outi, cnts, cloc):
    _sc_router_body(scores_hbm, biased_hbm, outv_hbm, outi_hbm, hist_hbm,
                    sblk, bblk, wrk, outv, outi, cnts, cloc)


def kernel(x, expert_bias, gate_w):
    scores3, biased3 = _gate_scores(x, gate_w, expert_bias)
    tv, ti, histp = _sc_router(scores3, biased3)
    return tv, ti, histp.sum(axis=0)
